# baseline (speedup 1.0000x reference)
import functools

import jax
import jax.numpy as jnp
from jax import lax
from jax.experimental import pallas as pl
from jax.experimental.pallas import tpu as pltpu
from jax.experimental.pallas import tpu_sc as plsc

R = 5
D = 128
DH = 64
NPAD = 5120
NR = R * NPAD
NSUB = 16
ROWS_PER_TILE = NR // NSUB
CHUNK = 128
TCH = 157
EDGES_PER_TILE = TCH * CHUNK
E_PAD = NSUB * EDGES_PER_TILE
JCH = 12
GROUPS = TCH // JCH
TAIL = TCH - GROUPS * JCH
GE = JCH * CHUNK
CW = 8

_mesh = plsc.VectorSubcoreMesh(core_axis_name="c", subcore_axis_name="s")


@functools.partial(
    pl.kernel,
    mesh=_mesh,
    compiler_params=pltpu.CompilerParams(use_tc_tiling_on_sc=False),
    out_type=[
        jax.ShapeDtypeStruct((4 * NR, DH), jnp.bfloat16),
        jax.ShapeDtypeStruct((4 * NR, CW), jnp.float32),
    ],
    scratch_types=[
        pltpu.VMEM_SHARED((NR, DH), jnp.bfloat16),
        pltpu.VMEM_SHARED((NR, CW), jnp.float32),
        pltpu.VMEM((GE,), jnp.int32),
        pltpu.VMEM((GE,), jnp.int32),
        pltpu.VMEM((GE,), jnp.int32),
        pltpu.VMEM((JCH, CHUNK), jnp.int32),
        pltpu.VMEM((JCH, CHUNK), jnp.int32),
        pltpu.VMEM((JCH, CHUNK, DH), jnp.bfloat16),
        pltpu.VMEM((CHUNK, CW), jnp.float32),
        pltpu.VMEM((320, CW), jnp.float32),
        pltpu.SemaphoreType.DMA,
        pltpu.SemaphoreType.DMA,
        pltpu.SemaphoreType.DMA,
    ],
)
def _sc_accumulate(xv_tab, xu_tab, us, vs, rt, zrow, z8, o8, s_out, cnt_out,
                   acc, cnt2, ubuf, vbuf, rbuf, dstb, srcb, rows, ones2,
                   zc, semg, sems, semc):
    if True:
        c = lax.axis_index("c")
        s = lax.axis_index("s")
        r0 = s * ROWS_PER_TILE
        coff = c * NPAD
        nzfull = ROWS_PER_TILE // CHUNK
        nzrem = ROWS_PER_TILE % CHUNK

        pltpu.sync_copy(z8, zc)
        pltpu.sync_copy(o8, ones2)

        for d in range(2):
            dst_hbm = us if d == 0 else vs
            src_hbm = vs if d == 0 else us
            table = xv_tab if d == 0 else xu_tab

            pltpu.sync_copy(zrow, rows.at[0])
            for k in range(nzfull):
                pltpu.sync_copy(rows.at[0],
                                acc.at[pl.ds(r0 + k * CHUNK, CHUNK), :])
            if nzrem:
                pltpu.sync_copy(
                    rows.at[0].at[pl.ds(0, nzrem), :],
                    acc.at[pl.ds(r0 + nzfull * CHUNK, nzrem), :],
                )
            for k in range(5):
                pltpu.sync_copy(zc, cnt2.at[pl.ds(r0 + k * 320, 320), :])
            plsc.subcore_barrier()

            def block(base, nch):
                ne = nch * CHUNK
                pltpu.sync_copy(dst_hbm.at[pl.ds(base, ne)],
                                ubuf.at[pl.ds(0, ne)])
                pltpu.sync_copy(src_hbm.at[pl.ds(base, ne)],
                                vbuf.at[pl.ds(0, ne)])
                pltpu.sync_copy(rt.at[pl.ds(base, ne)],
                                rbuf.at[pl.ds(0, ne)])
                for j in range(nch):
                    for i in range(CHUNK // 16):
                        sl = pl.ds(j * CHUNK + i * 16, 16)
                        osl = pl.ds(i * 16, 16)
                        dstb[j, osl] = rbuf[sl] * NPAD + ubuf[sl]
                        srcb[j, osl] = vbuf[sl] + coff
                gat = [
                    pltpu.async_copy(table.at[srcb.at[j]], rows.at[j], semg)
                    for j in range(nch)
                ]
                sca = []
                cnt_cp = []
                for j in range(nch):
                    gat[j].wait()
                    sca.append(
                        pltpu.async_copy(rows.at[j], acc.at[dstb.at[j]],
                                         sems, add=True))

                    @pl.when(c == (j % 2))
                    def _():
                        cnt_cp.append(
                            pltpu.async_copy(ones2, cnt2.at[dstb.at[j]],
                                             semc, add=True))

                for cp in sca:
                    cp.wait()
                for j in range(nch):

                    @pl.when(c == (j % 2))
                    def _():
                        cnt_cp[j].wait()

            def group(g, carry):
                block(s * EDGES_PER_TILE + g * GE, JCH)
                return carry

            lax.fori_loop(0, GROUPS, group, 0)
            if TAIL:
                block(s * EDGES_PER_TILE + GROUPS * GE, TAIL)
            plsc.subcore_barrier()

            pltpu.sync_copy(
                acc.at[pl.ds(r0, ROWS_PER_TILE), :],
                s_out.at[pl.ds((2 * d + c) * NR + r0, ROWS_PER_TILE), :],
            )
            pltpu.sync_copy(
                cnt2.at[pl.ds(r0, ROWS_PER_TILE), :],
                cnt_out.at[pl.ds((2 * d + c) * NR + r0, ROWS_PER_TILE), :],
            )


BN = 1024
NB = NPAD // BN


def _tc_body(s_ref, c_ref, w_ref, h_ref):
    x = s_ref[0]
    cc = c_ref[0]
    acc = jnp.zeros((BN, D), jnp.float32)
    for r in range(R):
        cnt = cc[r][:, :1] + cc[R + r][:, :1]
        inv = 1.0 / jnp.maximum(cnt, 1.0)
        xr = jnp.concatenate([x[r], x[R + r]],
                             axis=1).astype(jnp.float32) * inv
        acc = acc + jnp.dot(
            xr,
            w_ref[r],
            preferred_element_type=jnp.float32,
            precision=lax.Precision.HIGHEST,
        )
    h_ref[0] = acc


def kernel(x_u, x_v, W, u_s, v_s, rate):
    n_u = x_u.shape[0]
    n_v = x_v.shape[0]
    e = u_s.shape[0]

    xv_p = jnp.pad(x_v, ((0, NPAD - n_v), (0, 0))).astype(jnp.bfloat16)
    xu_p = jnp.pad(x_u, ((0, NPAD - n_u), (0, 0))).astype(jnp.bfloat16)
    xv_tab = jnp.concatenate([xv_p[:, :DH], xv_p[:, DH:]], axis=0)
    xu_tab = jnp.concatenate([xu_p[:, :DH], xu_p[:, DH:]], axis=0)

    padn = E_PAD - e
    trash = jnp.full((padn,), NPAD - 1, jnp.int32)
    us_p = jnp.concatenate([u_s, trash])
    vs_p = jnp.concatenate([v_s, trash])
    rt_p = jnp.concatenate([rate, jnp.zeros((padn,), jnp.int32)])

    zrow = jnp.zeros((CHUNK, DH), jnp.bfloat16)
    z8 = jnp.zeros((320, CW), jnp.float32)
    o8 = jnp.ones((CHUNK, CW), jnp.float32)
    s_flat, cnt_flat = _sc_accumulate(xv_tab, xu_tab, us_p, vs_p, rt_p,
                                      zrow, z8, o8)

    s4 = s_flat.reshape(2, 2 * R, NPAD, DH)
    c4 = cnt_flat.reshape(2, 2 * R, NPAD, CW)

    h = pl.pallas_call(
        _tc_body,
        grid=(2, NB),
        in_specs=[
            pl.BlockSpec((1, 2 * R, BN, DH), lambda d, n: (d, 0, n, 0)),
            pl.BlockSpec((1, 2 * R, BN, CW), lambda d, n: (d, 0, n, 0)),
            pl.BlockSpec((R, D, D), lambda d, n: (0, 0, 0)),
        ],
        out_specs=pl.BlockSpec((1, BN, D), lambda d, n: (d, n, 0)),
        out_shape=jax.ShapeDtypeStruct((2, NPAD, D), jnp.float32),
    )(s4, c4, W)

    return h[0, :n_u], h[1, :n_v]


# speedup vs baseline: 1.0749x; 1.0749x over previous
import functools

import jax
import jax.numpy as jnp
from jax import lax
from jax.experimental import pallas as pl
from jax.experimental.pallas import tpu as pltpu
from jax.experimental.pallas import tpu_sc as plsc

R = 5
D = 128
DH = 64
NPAD = 5120
NR = R * NPAD
NSUB = 16
ROWS_PER_TILE = NR // NSUB
CHUNK = 128
TCH = 157
EDGES_PER_TILE = TCH * CHUNK
E_PAD = NSUB * EDGES_PER_TILE
JCH = 13
GROUPS = TCH // JCH
TAIL = TCH - GROUPS * JCH
GE = JCH * CHUNK
CW = 8

_mesh = plsc.VectorSubcoreMesh(core_axis_name="c", subcore_axis_name="s")


@functools.partial(
    pl.kernel,
    mesh=_mesh,
    compiler_params=pltpu.CompilerParams(use_tc_tiling_on_sc=False),
    out_type=[
        jax.ShapeDtypeStruct((4 * NR, DH), jnp.bfloat16),
        jax.ShapeDtypeStruct((2 * NR, CW), jnp.float32),
    ],
    scratch_types=[
        pltpu.VMEM_SHARED((NR, DH), jnp.bfloat16),
        pltpu.VMEM_SHARED((NR, CW), jnp.float32),
        pltpu.VMEM((GE,), jnp.int32),
        pltpu.VMEM((GE,), jnp.int32),
        pltpu.VMEM((GE,), jnp.int32),
        pltpu.VMEM((JCH, CHUNK), jnp.int32),
        pltpu.VMEM((JCH, CHUNK), jnp.int32),
        pltpu.VMEM((JCH, CHUNK, DH), jnp.bfloat16),
        pltpu.VMEM((CHUNK, CW), jnp.float32),
        pltpu.VMEM((320, CW), jnp.float32),
        pltpu.SemaphoreType.DMA,
        pltpu.SemaphoreType.DMA,
        pltpu.SemaphoreType.DMA,
    ],
)
def _sc_accumulate(xv_tab, xu_tab, us, vs, rt, zrow, z8, o8, s_out, cnt_out,
                   acc, cnt2, ubuf, vbuf, rbuf, dstb, srcb, rows, ones2,
                   zc, semg, sems, semc):
    if True:
        c = lax.axis_index("c")
        s = lax.axis_index("s")
        r0 = s * ROWS_PER_TILE
        coff = c * NPAD
        nzfull = ROWS_PER_TILE // CHUNK
        nzrem = ROWS_PER_TILE % CHUNK

        pltpu.sync_copy(z8, zc)
        pltpu.sync_copy(o8, ones2)

        for k in range(5):
            pltpu.sync_copy(zc, cnt2.at[pl.ds(r0 + k * 320, 320), :])

        for d in range(2):
            dst_hbm = us if d == 0 else vs
            src_hbm = vs if d == 0 else us
            table = xv_tab if d == 0 else xu_tab

            pltpu.sync_copy(zrow, rows.at[0])
            for k in range(nzfull):
                pltpu.sync_copy(rows.at[0],
                                acc.at[pl.ds(r0 + k * CHUNK, CHUNK), :])
            if nzrem:
                pltpu.sync_copy(
                    rows.at[0].at[pl.ds(0, nzrem), :],
                    acc.at[pl.ds(r0 + nzfull * CHUNK, nzrem), :],
                )
            plsc.subcore_barrier()

            def block(base, nch):
                ne = nch * CHUNK
                pltpu.sync_copy(dst_hbm.at[pl.ds(base, ne)],
                                ubuf.at[pl.ds(0, ne)])
                pltpu.sync_copy(src_hbm.at[pl.ds(base, ne)],
                                vbuf.at[pl.ds(0, ne)])
                pltpu.sync_copy(rt.at[pl.ds(base, ne)],
                                rbuf.at[pl.ds(0, ne)])
                for j in range(nch):
                    for i in range(CHUNK // 16):
                        sl = pl.ds(j * CHUNK + i * 16, 16)
                        osl = pl.ds(i * 16, 16)
                        dstb[j, osl] = rbuf[sl] * NPAD + ubuf[sl]
                        srcb[j, osl] = vbuf[sl] + coff
                gat = [
                    pltpu.async_copy(table.at[srcb.at[j]], rows.at[j], semg)
                    for j in range(nch)
                ]
                sca = []
                cnt_cp = []
                for j in range(nch):
                    gat[j].wait()
                    sca.append(
                        pltpu.async_copy(rows.at[j], acc.at[dstb.at[j]],
                                         sems, add=True))

                    @pl.when(c == d)
                    def _():
                        cnt_cp.append(
                            pltpu.async_copy(ones2, cnt2.at[dstb.at[j]],
                                             semc, add=True))

                for cp in sca:
                    cp.wait()

                @pl.when(c == d)
                def _():
                    for cp in cnt_cp:
                        cp.wait()

            def group(g, carry):
                block(s * EDGES_PER_TILE + g * GE, JCH)
                return carry

            lax.fori_loop(0, GROUPS, group, 0)
            if TAIL:
                block(s * EDGES_PER_TILE + GROUPS * GE, TAIL)
            plsc.subcore_barrier()

            pltpu.sync_copy(
                acc.at[pl.ds(r0, ROWS_PER_TILE), :],
                s_out.at[pl.ds((2 * d + c) * NR + r0, ROWS_PER_TILE), :],
            )

        pltpu.sync_copy(
            cnt2.at[pl.ds(r0, ROWS_PER_TILE), :],
            cnt_out.at[pl.ds(c * NR + r0, ROWS_PER_TILE), :],
        )


BN = 1024
NB = NPAD // BN


def _tc_body(s_ref, c_ref, w_ref, h_ref):
    x = s_ref[0]
    inv = 1.0 / jnp.maximum(c_ref[0][:, :, :1], 1.0)
    acc = jnp.zeros((BN, D), jnp.float32)
    for r in range(R):
        xr = jnp.concatenate([x[r], x[R + r]],
                             axis=1).astype(jnp.float32) * inv[r]
        acc = acc + jnp.dot(
            xr,
            w_ref[r],
            preferred_element_type=jnp.float32,
            precision=lax.Precision.HIGHEST,
        )
    h_ref[0] = acc


def kernel(x_u, x_v, W, u_s, v_s, rate):
    n_u = x_u.shape[0]
    n_v = x_v.shape[0]
    e = u_s.shape[0]

    xv_p = jnp.pad(x_v, ((0, NPAD - n_v), (0, 0))).astype(jnp.bfloat16)
    xu_p = jnp.pad(x_u, ((0, NPAD - n_u), (0, 0))).astype(jnp.bfloat16)
    xv_tab = jnp.concatenate([xv_p[:, :DH], xv_p[:, DH:]], axis=0)
    xu_tab = jnp.concatenate([xu_p[:, :DH], xu_p[:, DH:]], axis=0)

    padn = E_PAD - e
    trash = jnp.full((padn,), NPAD - 1, jnp.int32)
    us_p = jnp.concatenate([u_s, trash])
    vs_p = jnp.concatenate([v_s, trash])
    rt_p = jnp.concatenate([rate, jnp.zeros((padn,), jnp.int32)])

    zrow = jnp.zeros((CHUNK, DH), jnp.bfloat16)
    z8 = jnp.zeros((320, CW), jnp.float32)
    o8 = jnp.ones((CHUNK, CW), jnp.float32)
    s_flat, cnt_flat = _sc_accumulate(xv_tab, xu_tab, us_p, vs_p, rt_p,
                                      zrow, z8, o8)

    s4 = s_flat.reshape(2, 2 * R, NPAD, DH)
    c4 = cnt_flat.reshape(2, R, NPAD, CW)

    h = pl.pallas_call(
        _tc_body,
        grid=(2, NB),
        in_specs=[
            pl.BlockSpec((1, 2 * R, BN, DH), lambda d, n: (d, 0, n, 0)),
            pl.BlockSpec((1, R, BN, CW), lambda d, n: (d, 0, n, 0)),
            pl.BlockSpec((R, D, D), lambda d, n: (0, 0, 0)),
        ],
        out_specs=pl.BlockSpec((1, BN, D), lambda d, n: (d, n, 0)),
        out_shape=jax.ShapeDtypeStruct((2, NPAD, D), jnp.float32),
    )(s4, c4, W)

    return h[0, :n_u], h[1, :n_v]
